# Initial kernel scaffold; baseline (speedup 1.0000x reference)
import jax
import jax.numpy as jnp
from jax.experimental import pallas as pl


def kernel(y_cls, y_bbox, anchors):
    raise NotImplementedError("write your pallas kernel here")


# speedup vs baseline: 1.4357x; 1.4357x over previous
import math

import jax
import jax.numpy as jnp
from jax import lax
from jax.experimental import pallas as pl
from jax.experimental.pallas import tpu as pltpu

NCLS = 10
KPC = 50
KTOT = 100
RW, LN = 160, 128
NPAD = RW * LN
IOU_THR = 0.5
ANY_IOU = 0.7
LOGIT_THR = math.log(0.05 / 0.95)
NEG = float("-inf")
PADV = -1e30


def _pp_body(lg_ref, ty_ref, tx_ref, th_ref, tw_ref,
             a0_ref, a1_ref, a2_ref, a3_ref,
             oy1_ref, ox1_ref, oy2_ref, ox2_ref, os_ref, oc_ref, on_ref,
             by1_ref, bx1_ref, by2_ref, bx2_ref):
    f32 = jnp.float32
    a0 = a0_ref[...]
    a1 = a1_ref[...]
    a2 = a2_ref[...]
    a3 = a3_ref[...]
    ah = a2 - a0
    aw = a3 - a1
    ay = a0 + ah * 0.5
    ax = a1 + aw * 0.5
    ty = ty_ref[0]
    tx = tx_ref[0]
    th = th_ref[0]
    tw = tw_ref[0]
    cy = ty * ah + ay
    cx = tx * aw + ax
    h = jnp.exp(th) * ah
    w = jnp.exp(tw) * aw
    by1_ref[...] = cy - h / 2
    bx1_ref[...] = cx - w / 2
    by2_ref[...] = cy + h / 2
    bx2_ref[...] = cx + w / 2

    rowi = lax.broadcasted_iota(jnp.int32, (RW, LN), 0)
    coli = lax.broadcasted_iota(jnp.int32, (RW, LN), 1)
    flat = rowi * LN + coli
    r4 = lax.broadcasted_iota(jnp.int32, (4, LN), 0)
    c4 = lax.broadcasted_iota(jnp.int32, (4, LN), 1)
    flat4 = r4 * LN + c4
    cls4 = flat4 // KPC
    i128 = lax.broadcasted_iota(jnp.int32, (1, LN), 1)
    BIGI = jnp.int32(2**30)
    FIN = f32(-3e38)

    def run_class(c, cand):
        cy1, cx1, cy2, cx2, cs = cand
        s0 = lg_ref[0, c]

        def pick_step(k, st):
            s, cy1, cx1, cy2, cx2, cs = st
            by1 = by1_ref[...]
            bx1 = bx1_ref[...]
            by2 = by2_ref[...]
            bx2 = bx2_ref[...]
            m = jnp.max(s)
            idx = jnp.min(jnp.where(s == m, flat, BIGI))
            sel = flat == idx
            py1 = jnp.sum(jnp.where(sel, by1, 0.0))
            px1 = jnp.sum(jnp.where(sel, bx1, 0.0))
            py2 = jnp.sum(jnp.where(sel, by2, 0.0))
            px2 = jnp.sum(jnp.where(sel, bx2, 0.0))
            valid = m > f32(LOGIT_THR)
            yy1 = jnp.maximum(py1, by1)
            xx1 = jnp.maximum(px1, bx1)
            yy2 = jnp.minimum(py2, by2)
            xx2 = jnp.minimum(px2, bx2)
            inter = jnp.maximum(yy2 - yy1, 0.0) * jnp.maximum(xx2 - xx1, 0.0)
            a1_ = jnp.maximum(py2 - py1, 0.0) * jnp.maximum(px2 - px1, 0.0)
            a2_ = jnp.maximum(by2 - by1, 0.0) * jnp.maximum(bx2 - bx1, 0.0)
            iou = inter / (a1_ + a2_ - inter + 1e-8)
            s = jnp.where(iou >= IOU_THR, f32(NEG), s)
            slot = c * KPC + k
            at = flat4 == slot
            cs = jnp.where(at, jnp.where(valid, m, f32(NEG)), cs)
            wv = at & valid
            cy1 = jnp.where(wv, py1, cy1)
            cx1 = jnp.where(wv, px1, cx1)
            cy2 = jnp.where(wv, py2, cy2)
            cx2 = jnp.where(wv, px2, cx2)
            return (s, cy1, cx1, cy2, cx2, cs)

        out = lax.fori_loop(0, KPC, pick_step, (s0, cy1, cx1, cy2, cx2, cs))
        return out[1:]

    z4 = jnp.zeros((4, LN), f32)
    cand0 = (z4, z4, z4, z4, jnp.full((4, LN), NEG, f32))
    cy1, cx1, cy2, cx2, cs = lax.fori_loop(0, NCLS, run_class, cand0)

    def topk_step(t, st):
        cs, t_y1, t_x1, t_y2, t_x2, t_s, t_c = st
        m = jnp.max(cs)
        idx = jnp.min(jnp.where(cs == m, flat4, BIGI))
        sel4 = flat4 == idx
        tv = m > FIN
        vy1 = jnp.sum(jnp.where(sel4, cy1, 0.0))
        vx1 = jnp.sum(jnp.where(sel4, cx1, 0.0))
        vy2 = jnp.sum(jnp.where(sel4, cy2, 0.0))
        vx2 = jnp.sum(jnp.where(sel4, cx2, 0.0))
        vc = jnp.sum(jnp.where(sel4, cls4, 0))
        at = i128 == t
        t_s = jnp.where(at, m, t_s)
        wv = at & tv
        t_y1 = jnp.where(wv, vy1, t_y1)
        t_x1 = jnp.where(wv, vx1, t_x1)
        t_y2 = jnp.where(wv, vy2, t_y2)
        t_x2 = jnp.where(wv, vx2, t_x2)
        t_c = jnp.where(wv, vc, t_c)
        cs = jnp.where(sel4, f32(NEG), cs)
        return (cs, t_y1, t_x1, t_y2, t_x2, t_s, t_c)

    z1 = jnp.zeros((1, LN), f32)
    zi = jnp.zeros((1, LN), jnp.int32)
    st = (cs, z1, z1, z1, z1, jnp.full((1, LN), NEG, f32), zi)
    _, t_y1, t_x1, t_y2, t_x2, t_s, t_c = lax.fori_loop(0, KTOT, topk_step, st)

    def nms2_step(i, st):
        kept, oy1, ox1, oy2, ox2, osg, ocl, cnt = st
        sel = i128 == i
        s_i = jnp.sum(jnp.where(sel, t_s, 0.0))
        y1_i = jnp.sum(jnp.where(sel, t_y1, 0.0))
        x1_i = jnp.sum(jnp.where(sel, t_x1, 0.0))
        y2_i = jnp.sum(jnp.where(sel, t_y2, 0.0))
        x2_i = jnp.sum(jnp.where(sel, t_x2, 0.0))
        c_i = jnp.sum(jnp.where(sel, t_c, 0))
        valid = s_i > FIN
        yy1 = jnp.maximum(y1_i, t_y1)
        xx1 = jnp.maximum(x1_i, t_x1)
        yy2 = jnp.minimum(y2_i, t_y2)
        xx2 = jnp.minimum(x2_i, t_x2)
        inter = jnp.maximum(yy2 - yy1, 0.0) * jnp.maximum(xx2 - xx1, 0.0)
        a1_ = jnp.maximum(y2_i - y1_i, 0.0) * jnp.maximum(x2_i - x1_i, 0.0)
        a2_ = jnp.maximum(t_y2 - t_y1, 0.0) * jnp.maximum(t_x2 - t_x1, 0.0)
        iou = inter / (a1_ + a2_ - inter + 1e-8)
        sup = jnp.any((kept > 0) & (iou >= ANY_IOU))
        keep = valid & jnp.logical_not(sup)
        at = (i128 == cnt) & keep
        oy1 = jnp.where(at, y1_i, oy1)
        ox1 = jnp.where(at, x1_i, ox1)
        oy2 = jnp.where(at, y2_i, oy2)
        ox2 = jnp.where(at, x2_i, ox2)
        osg = jnp.where(at, s_i, osg)
        ocl = jnp.where(at, c_i, ocl)
        kept = jnp.where(sel & keep, 1, kept)
        cnt = jnp.where(keep, cnt + 1, cnt)
        return (kept, oy1, ox1, oy2, ox2, osg, ocl, cnt)

    st = (jnp.zeros((1, LN), jnp.int32), z1, z1, z1, z1, z1, zi,
          jnp.int32(0))
    _, oy1, ox1, oy2, ox2, osg, ocl, cnt = lax.fori_loop(0, KTOT, nms2_step, st)

    live = i128 < cnt
    oy1_ref[0] = oy1
    ox1_ref[0] = ox1
    oy2_ref[0] = oy2
    ox2_ref[0] = ox2
    os_ref[0] = jnp.where(live, jax.nn.sigmoid(osg), 0.0)
    oc_ref[0] = jnp.where(live, ocl, -1)
    on_ref[0] = jnp.zeros((1, LN), jnp.int32) + cnt


def kernel(y_cls, y_bbox, anchors):
    B, N, C = y_cls.shape
    pad = NPAD - N
    lg = jnp.transpose(y_cls, (0, 2, 1))
    lg = jnp.pad(lg, ((0, 0), (0, 0), (0, pad)),
                 constant_values=PADV).reshape(B, C, RW, LN)
    dl = [jnp.pad(y_bbox[..., i], ((0, 0), (0, pad))).reshape(B, RW, LN)
          for i in range(4)]
    al = [jnp.pad(anchors[:, i], (0, pad)).reshape(RW, LN) for i in range(4)]

    f32 = jnp.float32
    outs = pl.pallas_call(
        _pp_body,
        grid=(B,),
        in_specs=[pl.BlockSpec((1, C, RW, LN), lambda b: (b, 0, 0, 0))]
        + [pl.BlockSpec((1, RW, LN), lambda b: (b, 0, 0))] * 4
        + [pl.BlockSpec((RW, LN), lambda b: (0, 0))] * 4,
        out_specs=[pl.BlockSpec((1, 1, LN), lambda b: (b, 0, 0))] * 7,
        out_shape=[jax.ShapeDtypeStruct((B, 1, LN), f32)] * 5
        + [jax.ShapeDtypeStruct((B, 1, LN), jnp.int32)] * 2,
        scratch_shapes=[pltpu.VMEM((RW, LN), f32)] * 4,
    )(lg, *dl, *al)
    oy1, ox1, oy2, ox2, osd, ocd, ond = outs
    b = jnp.stack([oy1[:, 0, :KTOT], ox1[:, 0, :KTOT],
                   oy2[:, 0, :KTOT], ox2[:, 0, :KTOT]], axis=-1)
    return b, osd[:, 0, :KTOT], ocd[:, 0, :KTOT], ond[:, 0, 0]
